# Initial kernel scaffold; baseline (speedup 1.0000x reference)
import jax
import jax.numpy as jnp
from jax.experimental import pallas as pl


def kernel(p0, p1, p2, conv_w, conv_b, memory_bank):
    raise NotImplementedError("write your pallas kernel here")


# speedup vs baseline: 14.1195x; 14.1195x over previous
import functools

import jax
import jax.numpy as jnp
from jax.experimental import pallas as pl
from jax.experimental.pallas import tpu as pltpu

_ROWS_BLK = 256
_K = 1792
_KIN = 1920
_NCOLS = 3200
_CTILE = 640
_NTILES = _NCOLS // _CTILE
_BIG = 3.0e38
_PAD_DIST = 1.0e30


def _fused_kernel(xc_ref, w_ref, b_ref, mb_ref, out_ref, cent_ref):
    r = pl.program_id(0)

    @pl.when(r == 0)
    def _():
        for c in range(_NTILES):
            sl = pl.ds(c * _CTILE, _CTILE)
            t = mb_ref[:, sl]
            s = jnp.sum(t * t, axis=0, keepdims=True)
            ids = c * _CTILE + jax.lax.broadcasted_iota(
                jnp.int32, (1, _CTILE), 1)
            cent_ref[:, sl] = jnp.where(ids >= 3136, _PAD_DIST, s)

    phi = jnp.dot(xc_ref[...], w_ref[...],
                  preferred_element_type=jnp.float32) + b_ref[...]
    feat = jnp.sum(phi * phi, axis=1, keepdims=True)

    r0 = jnp.full((_ROWS_BLK, _CTILE), _BIG, jnp.float32)
    r1 = r0
    r2 = r0
    for c in range(_NTILES):
        sl = pl.ds(c * _CTILE, _CTILE)
        d = cent_ref[:, sl] - 2.0 * jnp.dot(
            phi, mb_ref[:, sl], preferred_element_type=jnp.float32)
        hi0 = jnp.maximum(r0, d)
        r0 = jnp.minimum(r0, d)
        hi1 = jnp.maximum(r1, hi0)
        r1 = jnp.minimum(r1, hi0)
        r2 = jnp.minimum(r2, hi1)

    iota = jax.lax.broadcasted_iota(jnp.int32, (_ROWS_BLK, _CTILE), 1)
    ds = []
    for _ in range(3):
        dmin = jnp.min(r0, axis=1, keepdims=True)
        sel = jnp.where(r0 == dmin, iota, jnp.int32(2 ** 30))
        li = jnp.min(sel, axis=1, keepdims=True)
        m = iota == li
        r0 = jnp.where(m, r1, r0)
        r1 = jnp.where(m, r2, r1)
        r2 = jnp.where(m, _BIG, r2)
        ds.append(dmin)

    d0, d1, d2 = [jnp.sqrt(jnp.maximum(feat + x, 1e-12)) for x in ds]
    score = d0 / (1.0 + jnp.exp(d0 - d1) + jnp.exp(d0 - d2))
    out_ref[...] = score


def _avg_pool3(x):
    s = jax.lax.reduce_window(x, 0.0, jax.lax.add, (1, 1, 3, 3),
                              (1, 1, 1, 1), 'SAME')
    return s / 9.0


@jax.jit
def kernel(p0, p1, p2, conv_w, conv_b, memory_bank):
    B = p0.shape[0]
    H = p0.shape[2]
    HW = H * H
    rows = B * HW

    sample = _avg_pool3(p0)
    o1 = jax.image.resize(_avg_pool3(p1), (B, p1.shape[1], H, H),
                          method='bilinear')
    o2 = jax.image.resize(_avg_pool3(p2), (B, p2.shape[1], H, H),
                          method='bilinear')
    sample = jnp.concatenate([sample, o1, o2], axis=1)
    t = sample.reshape(B, _K, HW).transpose(0, 2, 1)

    lin = jnp.linspace(-1.0, 1.0, H, dtype=jnp.float32)
    xx = jnp.tile(lin, H)
    yy = jnp.repeat(lin, H)
    coords = jnp.stack([xx, yy], axis=1)
    coords = jnp.broadcast_to(coords[None], (B, HW, 2))
    pad = jnp.zeros((B, HW, _KIN - _K - 2), jnp.float32)
    xc = jnp.concatenate([t, coords, pad], axis=2).reshape(rows, _KIN)

    w_t = jnp.concatenate(
        [conv_w.T, jnp.zeros((_KIN - _K - 2, _K), conv_w.dtype)], axis=0)
    b_row = conv_b.reshape(1, _K)
    mb = jnp.pad(memory_bank, ((0, 0), (0, _NCOLS - 3136)))

    grid = (rows // _ROWS_BLK,)
    score = pl.pallas_call(
        _fused_kernel,
        grid=grid,
        in_specs=[
            pl.BlockSpec((_ROWS_BLK, _KIN), lambda r: (r, 0)),
            pl.BlockSpec((_KIN, _K), lambda r: (0, 0)),
            pl.BlockSpec((1, _K), lambda r: (0, 0)),
            pl.BlockSpec((_K, _NCOLS), lambda r: (0, 0)),
        ],
        out_specs=pl.BlockSpec((_ROWS_BLK, 1), lambda r: (r, 0)),
        out_shape=jax.ShapeDtypeStruct((rows, 1), jnp.float32),
        scratch_shapes=[pltpu.VMEM((1, _NCOLS), jnp.float32)],
    )(xc, w_t, b_row, mb)

    score = score.reshape(B, H, H)[:, None, :, :]
    return (jnp.zeros(()), score)


# speedup vs baseline: 23.0310x; 1.6311x over previous
import jax
import jax.numpy as jnp
from jax.experimental import pallas as pl
from jax.experimental.pallas import tpu as pltpu

_RB = 448
_NRB = 7
_HW = 3136
_K = 1792
_NCOLS = 3200
_CTILE = 640
_NTILES = _NCOLS // _CTILE
_Q2 = 224
_BIG = 3.0e38
_PAD_DIST = 1.0e30


def _fused_kernel(t0_ref, q1_ref, q2_ref, k1_ref, k2_ref,
                  w0_ref, w1_ref, w2_ref, wxy_ref, b_ref, co_ref, mb_ref,
                  out_ref, cent_ref, h2_ref):
    b = pl.program_id(0)
    rb = pl.program_id(1)

    @pl.when(jnp.logical_and(b == 0, rb == 0))
    def _():
        for c in range(_NTILES):
            sl = pl.ds(c * _CTILE, _CTILE)
            t = mb_ref[:, sl].astype(jnp.float32)
            s = jnp.sum(t * t, axis=0, keepdims=True)
            ids = c * _CTILE + jax.lax.broadcasted_iota(
                jnp.int32, (1, _CTILE), 1)
            cent_ref[:, sl] = jnp.where(ids >= 3136, _PAD_DIST, s)

    @pl.when(rb == 0)
    def _():
        h2_ref[...] = jnp.dot(q2_ref[0], w2_ref[...],
                              preferred_element_type=jnp.float32
                              ).astype(jnp.bfloat16)

    phi = jnp.dot(t0_ref[0], w0_ref[...],
                  preferred_element_type=jnp.float32)
    up1 = jnp.dot(k1_ref[...], q1_ref[0],
                  preferred_element_type=jnp.float32).astype(jnp.bfloat16)
    phi += jnp.dot(up1, w1_ref[...], preferred_element_type=jnp.float32)
    phi += jnp.dot(k2_ref[...], h2_ref[...],
                   preferred_element_type=jnp.float32)
    phi += co_ref[:, 0:1] * wxy_ref[0:1, :]
    phi += co_ref[:, 1:2] * wxy_ref[1:2, :]
    phi += b_ref[...]

    feat = jnp.sum(phi * phi, axis=1, keepdims=True)
    phib = phi.astype(jnp.bfloat16)

    r0 = jnp.full((_RB, _CTILE), _BIG, jnp.float32)
    r1 = r0
    r2 = r0
    for c in range(_NTILES):
        sl = pl.ds(c * _CTILE, _CTILE)
        d = cent_ref[:, sl] - 2.0 * jnp.dot(
            phib, mb_ref[:, sl], preferred_element_type=jnp.float32)
        hi0 = jnp.maximum(r0, d)
        r0 = jnp.minimum(r0, d)
        hi1 = jnp.maximum(r1, hi0)
        r1 = jnp.minimum(r1, hi0)
        r2 = jnp.minimum(r2, hi1)

    iota = jax.lax.broadcasted_iota(jnp.int32, (_RB, _CTILE), 1)
    ds = []
    for _ in range(3):
        dmin = jnp.min(r0, axis=1, keepdims=True)
        sel = jnp.where(r0 == dmin, iota, jnp.int32(2 ** 30))
        li = jnp.min(sel, axis=1, keepdims=True)
        m = iota == li
        r0 = jnp.where(m, r1, r0)
        r1 = jnp.where(m, r2, r1)
        r2 = jnp.where(m, _BIG, r2)
        ds.append(dmin)

    d0, d1, d2 = [jnp.sqrt(jnp.maximum(feat + x, 1e-12)) for x in ds]
    score = d0 / (1.0 + jnp.exp(d0 - d1) + jnp.exp(d0 - d2))
    out_ref[...] = score[None]


def _resize_mat(n_in, n_out):
    eye = jnp.eye(n_in, dtype=jnp.float32)
    return jax.image.resize(eye, (n_out, n_in), method='bilinear')


def _pool_mat(n):
    idx = jnp.arange(n)
    return (jnp.abs(idx[:, None] - idx[None, :]) <= 1).astype(jnp.float32) / 3.0


@jax.jit
def kernel(p0, p1, p2, conv_w, conv_b, memory_bank):
    B = p0.shape[0]
    H = p0.shape[2]
    f32, bf16 = jnp.float32, jnp.bfloat16

    q0 = p0.transpose(0, 2, 3, 1)
    t0 = jax.lax.reduce_window(q0, 0.0, jax.lax.add, (1, 3, 3, 1),
                               (1, 1, 1, 1), 'SAME') / 9.0
    t0 = t0.reshape(B, _HW, 256).astype(bf16)

    q1 = p1.transpose(0, 2, 3, 1).reshape(B, 784, 512).astype(bf16)
    q2 = p2.transpose(0, 2, 3, 1).reshape(B, 196, 1024)
    q2 = jnp.pad(q2, ((0, 0), (0, _Q2 - 196), (0, 0))).astype(bf16)

    g1 = _resize_mat(28, H) @ _pool_mat(28)
    g2 = _resize_mat(14, H) @ _pool_mat(14)
    k1 = jnp.kron(g1, g1).astype(bf16)
    k2 = jnp.kron(g2, g2)
    k2 = jnp.pad(k2, ((0, 0), (0, _Q2 - 196))).astype(bf16)

    wt = conv_w.T
    w0 = wt[0:256].astype(bf16)
    w1 = wt[256:768].astype(bf16)
    w2 = wt[768:1792].astype(bf16)
    wxy = wt[1792:1794].astype(f32)
    b_row = conv_b.reshape(1, _K).astype(f32)

    lin = jnp.linspace(-1.0, 1.0, H, dtype=f32)
    coords = jnp.stack([jnp.tile(lin, H), jnp.repeat(lin, H)], axis=1)

    mb = jnp.pad(memory_bank, ((0, 0), (0, _NCOLS - 3136))).astype(bf16)

    grid = (B, _NRB)
    score = pl.pallas_call(
        _fused_kernel,
        grid=grid,
        in_specs=[
            pl.BlockSpec((1, _RB, 256), lambda b, r: (b, r, 0)),
            pl.BlockSpec((1, 784, 512), lambda b, r: (b, 0, 0)),
            pl.BlockSpec((1, _Q2, 1024), lambda b, r: (b, 0, 0)),
            pl.BlockSpec((_RB, 784), lambda b, r: (r, 0)),
            pl.BlockSpec((_RB, _Q2), lambda b, r: (r, 0)),
            pl.BlockSpec((256, _K), lambda b, r: (0, 0)),
            pl.BlockSpec((512, _K), lambda b, r: (0, 0)),
            pl.BlockSpec((1024, _K), lambda b, r: (0, 0)),
            pl.BlockSpec((2, _K), lambda b, r: (0, 0)),
            pl.BlockSpec((1, _K), lambda b, r: (0, 0)),
            pl.BlockSpec((_RB, 2), lambda b, r: (r, 0)),
            pl.BlockSpec((_K, _NCOLS), lambda b, r: (0, 0)),
        ],
        out_specs=pl.BlockSpec((1, _RB, 1), lambda b, r: (b, r, 0)),
        out_shape=jax.ShapeDtypeStruct((B, _HW, 1), f32),
        scratch_shapes=[
            pltpu.VMEM((1, _NCOLS), f32),
            pltpu.VMEM((_Q2, _K), bf16),
        ],
    )(t0, q1, q2, k1, k2, w0, w1, w2, wxy, b_row, coords, mb)

    score = score.reshape(B, H, H)[:, None, :, :]
    return (jnp.zeros(()), score)


# speedup vs baseline: 23.5343x; 1.0219x over previous
import jax
import jax.numpy as jnp
from jax.experimental import pallas as pl
from jax.experimental.pallas import tpu as pltpu

_RB = 448
_NRB = 7
_HW = 3136
_K = 1792
_NCOLS = 3200
_CTILE = 640
_NTILES = _NCOLS // _CTILE
_Q2 = 224
_BIG = 3.0e38
_PAD_DIST = 1.0e30


def _fused_kernel(t0_ref, q1_ref, q2_ref, k1_ref, k2_ref,
                  w0_ref, w1_ref, w2_ref, wxy_ref, b_ref, co_ref, mb_ref,
                  out_ref, cent_ref, h2_ref):
    b = pl.program_id(0)
    rb = pl.program_id(1)

    @pl.when(jnp.logical_and(b == 0, rb == 0))
    def _():
        for c in range(_NTILES):
            sl = pl.ds(c * _CTILE, _CTILE)
            t = mb_ref[:, sl].astype(jnp.float32)
            s = jnp.sum(t * t, axis=0, keepdims=True)
            ids = c * _CTILE + jax.lax.broadcasted_iota(
                jnp.int32, (1, _CTILE), 1)
            cent_ref[:, sl] = jnp.where(ids >= 3136, _PAD_DIST, s)

    @pl.when(rb == 0)
    def _():
        h2_ref[...] = jnp.dot(q2_ref[0], w2_ref[...],
                              preferred_element_type=jnp.float32
                              ).astype(jnp.bfloat16)

    phi = jnp.dot(t0_ref[0], w0_ref[...],
                  preferred_element_type=jnp.float32)
    up1 = jnp.dot(k1_ref[...], q1_ref[0],
                  preferred_element_type=jnp.float32).astype(jnp.bfloat16)
    phi += jnp.dot(up1, w1_ref[...], preferred_element_type=jnp.float32)
    phi += jnp.dot(k2_ref[...], h2_ref[...],
                   preferred_element_type=jnp.float32)
    phi += co_ref[:, 0:1] * wxy_ref[0:1, :]
    phi += co_ref[:, 1:2] * wxy_ref[1:2, :]
    phi += b_ref[...]

    feat = jnp.sum(phi * phi, axis=1, keepdims=True)
    phib = (2.0 * phi).astype(jnp.bfloat16)

    r0 = jnp.full((_RB, 128), _BIG, jnp.float32)
    r1 = r0
    r2 = r0
    for c in range(_NTILES):
        sl = pl.ds(c * _CTILE, _CTILE)
        d = cent_ref[:, sl] - jnp.dot(
            phib, mb_ref[:, sl], preferred_element_type=jnp.float32)
        for s in range(_CTILE // 128):
            ds_ = d[:, s * 128:(s + 1) * 128]
            hi0 = jnp.maximum(r0, ds_)
            r0 = jnp.minimum(r0, ds_)
            hi1 = jnp.maximum(r1, hi0)
            r1 = jnp.minimum(r1, hi0)
            r2 = jnp.minimum(r2, hi1)

    iota = jax.lax.broadcasted_iota(jnp.int32, (_RB, 128), 1)
    ds = []
    for _ in range(3):
        dmin = jnp.min(r0, axis=1, keepdims=True)
        sel = jnp.where(r0 == dmin, iota, jnp.int32(2 ** 30))
        li = jnp.min(sel, axis=1, keepdims=True)
        m = iota == li
        r0 = jnp.where(m, r1, r0)
        r1 = jnp.where(m, r2, r1)
        r2 = jnp.where(m, _BIG, r2)
        ds.append(dmin)

    d0, d1, d2 = [jnp.sqrt(jnp.maximum(feat + x, 1e-12)) for x in ds]
    score = d0 / (1.0 + jnp.exp(d0 - d1) + jnp.exp(d0 - d2))
    out_ref[...] = score[None]


def _resize_mat(n_in, n_out):
    eye = jnp.eye(n_in, dtype=jnp.float32)
    return jax.image.resize(eye, (n_out, n_in), method='bilinear')


def _pool_mat(n):
    idx = jnp.arange(n)
    return (jnp.abs(idx[:, None] - idx[None, :]) <= 1).astype(jnp.float32) / 3.0


@jax.jit
def kernel(p0, p1, p2, conv_w, conv_b, memory_bank):
    B = p0.shape[0]
    H = p0.shape[2]
    f32, bf16 = jnp.float32, jnp.bfloat16

    q0 = p0.transpose(0, 2, 3, 1)
    t0 = jax.lax.reduce_window(q0, 0.0, jax.lax.add, (1, 3, 3, 1),
                               (1, 1, 1, 1), 'SAME') / 9.0
    t0 = t0.reshape(B, _HW, 256).astype(bf16)

    q1 = p1.transpose(0, 2, 3, 1).reshape(B, 784, 512).astype(bf16)
    q2 = p2.transpose(0, 2, 3, 1).reshape(B, 196, 1024)
    q2 = jnp.pad(q2, ((0, 0), (0, _Q2 - 196), (0, 0))).astype(bf16)

    g1 = _resize_mat(28, H) @ _pool_mat(28)
    g2 = _resize_mat(14, H) @ _pool_mat(14)
    k1 = jnp.kron(g1, g1).astype(bf16)
    k2 = jnp.kron(g2, g2)
    k2 = jnp.pad(k2, ((0, 0), (0, _Q2 - 196))).astype(bf16)

    wt = conv_w.T
    w0 = wt[0:256].astype(bf16)
    w1 = wt[256:768].astype(bf16)
    w2 = wt[768:1792].astype(bf16)
    wxy = wt[1792:1794].astype(f32)
    b_row = conv_b.reshape(1, _K).astype(f32)

    lin = jnp.linspace(-1.0, 1.0, H, dtype=f32)
    coords = jnp.stack([jnp.tile(lin, H), jnp.repeat(lin, H)], axis=1)

    mb = jnp.pad(memory_bank, ((0, 0), (0, _NCOLS - 3136))).astype(bf16)

    grid = (B, _NRB)
    score = pl.pallas_call(
        _fused_kernel,
        grid=grid,
        in_specs=[
            pl.BlockSpec((1, _RB, 256), lambda b, r: (b, r, 0)),
            pl.BlockSpec((1, 784, 512), lambda b, r: (b, 0, 0)),
            pl.BlockSpec((1, _Q2, 1024), lambda b, r: (b, 0, 0)),
            pl.BlockSpec((_RB, 784), lambda b, r: (r, 0)),
            pl.BlockSpec((_RB, _Q2), lambda b, r: (r, 0)),
            pl.BlockSpec((256, _K), lambda b, r: (0, 0)),
            pl.BlockSpec((512, _K), lambda b, r: (0, 0)),
            pl.BlockSpec((1024, _K), lambda b, r: (0, 0)),
            pl.BlockSpec((2, _K), lambda b, r: (0, 0)),
            pl.BlockSpec((1, _K), lambda b, r: (0, 0)),
            pl.BlockSpec((_RB, 2), lambda b, r: (r, 0)),
            pl.BlockSpec((_K, _NCOLS), lambda b, r: (0, 0)),
        ],
        out_specs=pl.BlockSpec((1, _RB, 1), lambda b, r: (b, r, 0)),
        out_shape=jax.ShapeDtypeStruct((B, _HW, 1), f32),
        scratch_shapes=[
            pltpu.VMEM((1, _NCOLS), f32),
            pltpu.VMEM((_Q2, _K), bf16),
        ],
    )(t0, q1, q2, k1, k2, w0, w1, w2, wxy, b_row, coords, mb)

    score = score.reshape(B, H, H)[:, None, :, :]
    return (jnp.zeros(()), score)


# speedup vs baseline: 23.8110x; 1.0118x over previous
import jax
import jax.numpy as jnp
from jax.experimental import pallas as pl
from jax.experimental.pallas import tpu as pltpu

_RB = 448
_NRB = 7
_HW = 3136
_K = 1792
_NCOLS = 3200
_CTILE = 640
_NTILES = _NCOLS // _CTILE
_Q2 = 224
_KC = 1024
_KX = 256 + _KC
_BIG = 3.0e38
_PAD_DIST = 1.0e30


def _fused_kernel(t0_ref, kc_ref, q1_ref, q2_ref,
                  w0_ref, w1_ref, w2_ref, wxy_ref, b_ref, mb_ref,
                  out_ref, cent_ref, wcat_ref):
    b = pl.program_id(0)
    rb = pl.program_id(1)

    @pl.when(jnp.logical_and(b == 0, rb == 0))
    def _():
        for c in range(_NTILES):
            sl = pl.ds(c * _CTILE, _CTILE)
            t = mb_ref[:, sl].astype(jnp.float32)
            s = jnp.sum(t * t, axis=0, keepdims=True)
            ids = c * _CTILE + jax.lax.broadcasted_iota(
                jnp.int32, (1, _CTILE), 1)
            cent_ref[:, sl] = jnp.where(ids >= 3136, _PAD_DIST, s)
        wcat_ref[0:256, :] = w0_ref[...]
        wcat_ref[1264:1280, :] = jnp.concatenate(
            [wxy_ref[...], jnp.zeros((14, _K), jnp.bfloat16)], axis=0)

    @pl.when(rb == 0)
    def _():
        wcat_ref[256:1040, :] = jnp.dot(
            q1_ref[0], w1_ref[...],
            preferred_element_type=jnp.float32).astype(jnp.bfloat16)
        wcat_ref[1040:1264, :] = jnp.dot(
            q2_ref[0], w2_ref[...],
            preferred_element_type=jnp.float32).astype(jnp.bfloat16)

    x = jnp.concatenate([t0_ref[0], kc_ref[...]], axis=1)
    phi = jnp.dot(x, wcat_ref[...],
                  preferred_element_type=jnp.float32) + b_ref[...]

    feat = jnp.sum(phi * phi, axis=1, keepdims=True)
    phib = (2.0 * phi).astype(jnp.bfloat16)

    r0 = jnp.full((_RB, 128), _BIG, jnp.float32)
    r1 = r0
    r2 = r0
    for c in range(_NTILES):
        sl = pl.ds(c * _CTILE, _CTILE)
        d = cent_ref[:, sl] - jnp.dot(
            phib, mb_ref[:, sl], preferred_element_type=jnp.float32)
        for s in range(_CTILE // 128):
            ds_ = d[:, s * 128:(s + 1) * 128]
            hi0 = jnp.maximum(r0, ds_)
            r0 = jnp.minimum(r0, ds_)
            hi1 = jnp.maximum(r1, hi0)
            r1 = jnp.minimum(r1, hi0)
            r2 = jnp.minimum(r2, hi1)

    iota = jax.lax.broadcasted_iota(jnp.int32, (_RB, 128), 1)
    ds = []
    for _ in range(3):
        dmin = jnp.min(r0, axis=1, keepdims=True)
        sel = jnp.where(r0 == dmin, iota, jnp.int32(2 ** 30))
        li = jnp.min(sel, axis=1, keepdims=True)
        m = iota == li
        r0 = jnp.where(m, r1, r0)
        r1 = jnp.where(m, r2, r1)
        r2 = jnp.where(m, _BIG, r2)
        ds.append(dmin)

    d0, d1, d2 = [jnp.sqrt(jnp.maximum(feat + x_, 1e-12)) for x_ in ds]
    score = d0 / (1.0 + jnp.exp(d0 - d1) + jnp.exp(d0 - d2))
    out_ref[...] = score[None]


def _resize_mat(n_in, n_out):
    eye = jnp.eye(n_in, dtype=jnp.float32)
    return jax.image.resize(eye, (n_out, n_in), method='bilinear')


def _pool_mat(n):
    idx = jnp.arange(n)
    return (jnp.abs(idx[:, None] - idx[None, :]) <= 1).astype(jnp.float32) / 3.0


@jax.jit
def kernel(p0, p1, p2, conv_w, conv_b, memory_bank):
    B = p0.shape[0]
    H = p0.shape[2]
    f32, bf16 = jnp.float32, jnp.bfloat16

    q0 = p0.transpose(0, 2, 3, 1)
    t0 = jax.lax.reduce_window(q0, 0.0, jax.lax.add, (1, 3, 3, 1),
                               (1, 1, 1, 1), 'SAME') / 9.0
    t0 = t0.reshape(B, _HW, 256).astype(bf16)

    q1 = p1.transpose(0, 2, 3, 1).reshape(B, 784, 512).astype(bf16)
    q2 = p2.transpose(0, 2, 3, 1).reshape(B, 196, 1024)
    q2 = jnp.pad(q2, ((0, 0), (0, _Q2 - 196), (0, 0))).astype(bf16)

    g1 = _resize_mat(28, H) @ _pool_mat(28)
    g2 = _resize_mat(14, H) @ _pool_mat(14)
    k1 = jnp.kron(g1, g1)
    k2 = jnp.kron(g2, g2)

    lin = jnp.linspace(-1.0, 1.0, H, dtype=f32)
    coords = jnp.stack([jnp.tile(lin, H), jnp.repeat(lin, H)], axis=1)

    kc = jnp.concatenate(
        [k1, k2, jnp.zeros((_HW, _Q2 - 196), f32), coords,
         jnp.zeros((_HW, _KC - 784 - _Q2 - 2), f32)], axis=1).astype(bf16)

    wt = conv_w.T
    w0 = wt[0:256].astype(bf16)
    w1 = wt[256:768].astype(bf16)
    w2 = wt[768:1792].astype(bf16)
    wxy = wt[1792:1794].astype(bf16)
    b_row = conv_b.reshape(1, _K).astype(f32)

    mb = jnp.pad(memory_bank, ((0, 0), (0, _NCOLS - 3136))).astype(bf16)

    grid = (B, _NRB)
    score = pl.pallas_call(
        _fused_kernel,
        grid=grid,
        in_specs=[
            pl.BlockSpec((1, _RB, 256), lambda b, r: (b, r, 0)),
            pl.BlockSpec((_RB, _KC), lambda b, r: (r, 0)),
            pl.BlockSpec((1, 784, 512), lambda b, r: (b, 0, 0)),
            pl.BlockSpec((1, _Q2, 1024), lambda b, r: (b, 0, 0)),
            pl.BlockSpec((256, _K), lambda b, r: (0, 0)),
            pl.BlockSpec((512, _K), lambda b, r: (0, 0)),
            pl.BlockSpec((1024, _K), lambda b, r: (0, 0)),
            pl.BlockSpec((2, _K), lambda b, r: (0, 0)),
            pl.BlockSpec((1, _K), lambda b, r: (0, 0)),
            pl.BlockSpec((_K, _NCOLS), lambda b, r: (0, 0)),
        ],
        out_specs=pl.BlockSpec((1, _RB, 1), lambda b, r: (b, r, 0)),
        out_shape=jax.ShapeDtypeStruct((B, _HW, 1), f32),
        scratch_shapes=[
            pltpu.VMEM((1, _NCOLS), f32),
            pltpu.VMEM((_KX, _K), bf16),
        ],
    )(t0, kc, q1, q2, w0, w1, w2, wxy, b_row, mb)

    score = score.reshape(B, H, H)[:, None, :, :]
    return (jnp.zeros(()), score)


# speedup vs baseline: 26.8888x; 1.1293x over previous
import jax
import jax.numpy as jnp
import numpy as np
from jax.experimental import pallas as pl
from jax.experimental.pallas import tpu as pltpu

_RB = 448
_NRB = 7
_HW = 3136
_H = 56
_K = 1792
_NCOLS = 3136
_TILES = (640, 640, 640, 640, 576)
_Q2 = 224
_KC = 1024
_KX = 256 + _KC
_BIG = 3.0e38


def _build_kc():
    def resize_mat(n_in):
        eye = jnp.eye(n_in, dtype=jnp.float32)
        return jax.image.resize(eye, (_H, n_in), method='bilinear')

    def pool_mat(n):
        idx = np.arange(n)
        return ((np.abs(idx[:, None] - idx[None, :]) <= 1) / 3.0).astype(
            np.float32)

    g1 = np.asarray(resize_mat(28)) @ pool_mat(28)
    g2 = np.asarray(resize_mat(14)) @ pool_mat(14)
    k1 = np.kron(g1, g1)
    k2 = np.kron(g2, g2)
    lin = np.linspace(-1.0, 1.0, _H, dtype=np.float32)
    coords = np.stack([np.tile(lin, _H), np.repeat(lin, _H)], axis=1)
    kc = np.concatenate(
        [k1, np.pad(k2, ((0, 0), (0, _Q2 - 196))), coords,
         np.zeros((_HW, _KC - 784 - _Q2 - 2), np.float32)], axis=1)
    return np.asarray(jnp.asarray(kc).astype(jnp.bfloat16))


_KC_CONST = _build_kc()


def _fused_kernel(t0_ref, kc_ref, q1_ref, q2_ref,
                  w0_ref, w1_ref, w2_ref, wxy_ref, b_ref, mb_ref,
                  out_ref, cent_ref, wcat_ref):
    b = pl.program_id(0)
    rb = pl.program_id(1)

    @pl.when(jnp.logical_and(b == 0, rb == 0))
    def _():
        off = 0
        for w in _TILES:
            sl = pl.ds(off, w)
            t = mb_ref[:, sl].astype(jnp.float32)
            cent_ref[:, sl] = jnp.sum(t * t, axis=0, keepdims=True)
            off += w
        wcat_ref[0:256, :] = w0_ref[...]
        wcat_ref[1264:1280, :] = jnp.concatenate(
            [wxy_ref[...], jnp.zeros((14, _K), jnp.bfloat16)], axis=0)

    @pl.when(rb == 0)
    def _():
        wcat_ref[256:1040, :] = jnp.dot(
            q1_ref[0], w1_ref[...],
            preferred_element_type=jnp.float32).astype(jnp.bfloat16)
        wcat_ref[1040:1264, :] = jnp.dot(
            q2_ref[0], w2_ref[...],
            preferred_element_type=jnp.float32).astype(jnp.bfloat16)

    x = jnp.concatenate([t0_ref[0], kc_ref[...]], axis=1)
    phi = jnp.dot(x, wcat_ref[...],
                  preferred_element_type=jnp.float32) + b_ref[...]

    feat = jnp.sum(phi * phi, axis=1, keepdims=True)
    phib = (2.0 * phi).astype(jnp.bfloat16)

    r0 = jnp.full((_RB, 128), _BIG, jnp.float32)
    r1 = r0
    r2 = r0
    off = 0
    for w in _TILES:
        sl = pl.ds(off, w)
        d = cent_ref[:, sl] - jnp.dot(
            phib, mb_ref[:, sl], preferred_element_type=jnp.float32)
        off += w
        for s in range(0, w, 128):
            ds_ = d[:, s:s + 128]
            if ds_.shape[1] < 128:
                ds_ = jnp.concatenate(
                    [ds_, jnp.full((_RB, 128 - ds_.shape[1]), _BIG,
                                   jnp.float32)], axis=1)
            hi0 = jnp.maximum(r0, ds_)
            r0 = jnp.minimum(r0, ds_)
            hi1 = jnp.maximum(r1, hi0)
            r1 = jnp.minimum(r1, hi0)
            r2 = jnp.minimum(r2, hi1)

    iota = jax.lax.broadcasted_iota(jnp.int32, (_RB, 128), 1)
    ds = []
    for _ in range(3):
        dmin = jnp.min(r0, axis=1, keepdims=True)
        sel = jnp.where(r0 == dmin, iota, jnp.int32(2 ** 30))
        li = jnp.min(sel, axis=1, keepdims=True)
        m = iota == li
        r0 = jnp.where(m, r1, r0)
        r1 = jnp.where(m, r2, r1)
        r2 = jnp.where(m, _BIG, r2)
        ds.append(dmin)

    d0, d1, d2 = [jnp.sqrt(jnp.maximum(feat + x_, 1e-12)) for x_ in ds]
    score = d0 / (1.0 + jnp.exp(d0 - d1) + jnp.exp(d0 - d2))
    out_ref[...] = score[None]


@jax.jit
def kernel(p0, p1, p2, conv_w, conv_b, memory_bank):
    B = p0.shape[0]
    f32, bf16 = jnp.float32, jnp.bfloat16

    q0 = p0.transpose(0, 2, 3, 1)
    t0 = jax.lax.reduce_window(q0, 0.0, jax.lax.add, (1, 3, 3, 1),
                               (1, 1, 1, 1), 'SAME') / 9.0
    t0 = t0.reshape(B, _HW, 256).astype(bf16)

    q1 = p1.transpose(0, 2, 3, 1).reshape(B, 784, 512).astype(bf16)
    q2 = p2.transpose(0, 2, 3, 1).reshape(B, 196, 1024)
    q2 = jnp.pad(q2, ((0, 0), (0, _Q2 - 196), (0, 0))).astype(bf16)

    kc = jnp.asarray(_KC_CONST)

    wt = conv_w.T
    w0 = wt[0:256].astype(bf16)
    w1 = wt[256:768].astype(bf16)
    w2 = wt[768:1792].astype(bf16)
    wxy = wt[1792:1794].astype(bf16)
    b_row = conv_b.reshape(1, _K).astype(f32)

    mb = memory_bank.astype(bf16)

    grid = (B, _NRB)
    score = pl.pallas_call(
        _fused_kernel,
        grid=grid,
        in_specs=[
            pl.BlockSpec((1, _RB, 256), lambda b, r: (b, r, 0)),
            pl.BlockSpec((_RB, _KC), lambda b, r: (r, 0)),
            pl.BlockSpec((1, 784, 512), lambda b, r: (b, 0, 0)),
            pl.BlockSpec((1, _Q2, 1024), lambda b, r: (b, 0, 0)),
            pl.BlockSpec((256, _K), lambda b, r: (0, 0)),
            pl.BlockSpec((512, _K), lambda b, r: (0, 0)),
            pl.BlockSpec((1024, _K), lambda b, r: (0, 0)),
            pl.BlockSpec((2, _K), lambda b, r: (0, 0)),
            pl.BlockSpec((1, _K), lambda b, r: (0, 0)),
            pl.BlockSpec((_K, _NCOLS), lambda b, r: (0, 0)),
        ],
        out_specs=pl.BlockSpec((1, _RB, 1), lambda b, r: (b, r, 0)),
        out_shape=jax.ShapeDtypeStruct((B, _HW, 1), f32),
        scratch_shapes=[
            pltpu.VMEM((1, _NCOLS), f32),
            pltpu.VMEM((_KX, _K), jnp.bfloat16),
        ],
    )(t0, kc, q1, q2, w0, w1, w2, wxy, b_row, mb)

    score = score.reshape(B, _H, _H)[:, None, :, :]
    return (jnp.zeros(()), score)
